# baseline (speedup 1.0000x reference)
import functools

import jax
import jax.numpy as jnp
from jax.experimental import pallas as pl
from jax.experimental.pallas import tpu as pltpu

_N = 4194304
_COLS = 1024
_ROWS = _N // _COLS
_BROWS = 128
_GRID = _ROWS // _BROWS


def _tc_body(y_ref, s_ref, oy_ref, oys_ref, os_ref, accy, accys, accs):
    i = pl.program_id(0)
    yv = y_ref[...]
    sv = s_ref[...]
    ysel = jnp.where(sv == 1, yv, jnp.float32(0.0))
    sf = sv.astype(jnp.float32)
    py = jnp.sum(yv.reshape(-1, 8, _COLS), axis=0)
    pys = jnp.sum(ysel.reshape(-1, 8, _COLS), axis=0)
    ps = jnp.sum(sf.reshape(-1, 8, _COLS), axis=0)

    @pl.when(i == 0)
    def _init():
        accy[...] = py
        accys[...] = pys
        accs[...] = ps

    @pl.when(i != 0)
    def _acc():
        accy[...] += py
        accys[...] += pys
        accs[...] += ps

    @pl.when(i == _GRID - 1)
    def _fin():
        oy_ref[0, 0] = jnp.sum(accy[...])
        oys_ref[0, 0] = jnp.sum(accys[...])
        os_ref[0, 0] = jnp.sum(accs[...])


_tc_reduce = pl.pallas_call(
    _tc_body,
    grid=(_GRID,),
    in_specs=[
        pl.BlockSpec((_BROWS, _COLS), lambda i: (i, 0)),
        pl.BlockSpec((_BROWS, _COLS), lambda i: (i, 0)),
    ],
    out_specs=[
        pl.BlockSpec(memory_space=pltpu.SMEM),
        pl.BlockSpec(memory_space=pltpu.SMEM),
        pl.BlockSpec(memory_space=pltpu.SMEM),
    ],
    out_shape=[
        jax.ShapeDtypeStruct((1, 1), jnp.float32),
        jax.ShapeDtypeStruct((1, 1), jnp.float32),
        jax.ShapeDtypeStruct((1, 1), jnp.float32),
    ],
    scratch_shapes=[
        pltpu.VMEM((8, _COLS), jnp.float32),
        pltpu.VMEM((8, _COLS), jnp.float32),
        pltpu.VMEM((8, _COLS), jnp.float32),
    ],
    compiler_params=pltpu.CompilerParams(
        dimension_semantics=("arbitrary",),
    ),
)


def kernel(y_pred, s):
    y2 = y_pred.reshape(_ROWS, _COLS)
    s2 = s.reshape(_ROWS, _COLS)
    sy, sys_, cnt1 = _tc_reduce(y2, s2)
    sum_y = sy[0, 0]
    sum_ys = sys_[0, 0]
    c1 = cnt1[0, 0]
    c0 = jnp.float32(_N) - c1
    mean1 = sum_ys / c1
    mean0 = (sum_y - sum_ys) / c0
    return jnp.abs(mean0 - mean1)


# speedup vs baseline: 1.2223x; 1.2223x over previous
import functools

import jax
import jax.numpy as jnp
from jax.experimental import pallas as pl
from jax.experimental.pallas import tpu as pltpu

_N = 4194304
_COLS = 1024
_ROWS = _N // _COLS
_CHROWS = 128
_NCH = _ROWS // _CHROWS
_DEPTH = 4


def _tc_body(y_hbm, s_hbm, oy_ref, oys_ref, os_ref,
             ybuf, sbuf, sems_y, sems_s, accy, accys, accs):
    def start(k):
        slot = k % _DEPTH
        cy = pltpu.make_async_copy(
            y_hbm.at[pl.ds(k * _CHROWS, _CHROWS), :], ybuf.at[slot],
            sems_y.at[slot])
        cs = pltpu.make_async_copy(
            s_hbm.at[pl.ds(k * _CHROWS, _CHROWS), :], sbuf.at[slot],
            sems_s.at[slot])
        cy.start()
        cs.start()
        return cy, cs

    pending = [start(k) for k in range(_DEPTH)]
    accy[...] = jnp.zeros((8, _COLS), jnp.float32)
    accys[...] = jnp.zeros((8, _COLS), jnp.float32)
    accs[...] = jnp.zeros((8, _COLS), jnp.float32)

    for k in range(_NCH):
        slot = k % _DEPTH
        cy, cs = pending[slot]
        cy.wait()
        cs.wait()
        yv = ybuf[slot]
        sv = sbuf[slot]
        ysel = jnp.where(sv == 1, yv, jnp.float32(0.0))
        sf = sv.astype(jnp.float32)
        accy[...] += jnp.sum(yv.reshape(-1, 8, _COLS), axis=0)
        accys[...] += jnp.sum(ysel.reshape(-1, 8, _COLS), axis=0)
        accs[...] += jnp.sum(sf.reshape(-1, 8, _COLS), axis=0)
        if k + _DEPTH < _NCH:
            pending[slot] = start(k + _DEPTH)

    oy_ref[0, 0] = jnp.sum(accy[...])
    oys_ref[0, 0] = jnp.sum(accys[...])
    os_ref[0, 0] = jnp.sum(accs[...])


_tc_reduce = pl.pallas_call(
    _tc_body,
    in_specs=[
        pl.BlockSpec(memory_space=pl.ANY),
        pl.BlockSpec(memory_space=pl.ANY),
    ],
    out_specs=[
        pl.BlockSpec(memory_space=pltpu.SMEM),
        pl.BlockSpec(memory_space=pltpu.SMEM),
        pl.BlockSpec(memory_space=pltpu.SMEM),
    ],
    out_shape=[
        jax.ShapeDtypeStruct((1, 1), jnp.float32),
        jax.ShapeDtypeStruct((1, 1), jnp.float32),
        jax.ShapeDtypeStruct((1, 1), jnp.float32),
    ],
    scratch_shapes=[
        pltpu.VMEM((_DEPTH, _CHROWS, _COLS), jnp.float32),
        pltpu.VMEM((_DEPTH, _CHROWS, _COLS), jnp.int32),
        pltpu.SemaphoreType.DMA((_DEPTH,)),
        pltpu.SemaphoreType.DMA((_DEPTH,)),
        pltpu.VMEM((8, _COLS), jnp.float32),
        pltpu.VMEM((8, _COLS), jnp.float32),
        pltpu.VMEM((8, _COLS), jnp.float32),
    ],
)


def kernel(y_pred, s):
    y2 = y_pred.reshape(_ROWS, _COLS)
    s2 = s.reshape(_ROWS, _COLS)
    sy, sys_, cnt1 = _tc_reduce(y2, s2)
    sum_y = sy[0, 0]
    sum_ys = sys_[0, 0]
    c1 = cnt1[0, 0]
    c0 = jnp.float32(_N) - c1
    mean1 = sum_ys / c1
    mean0 = (sum_y - sum_ys) / c0
    return jnp.abs(mean0 - mean1)
